# baseline (speedup 1.0000x reference)
import functools

import jax
import jax.numpy as jnp
from jax import lax
from jax.experimental import pallas as pl
from jax.experimental.pallas import tpu as pltpu
from jax.experimental.pallas import tpu_sc as plsc

N_ACTIONS = 100000
A_DIM = 16
BATCH = 16384
HIST = 200
B_TOTAL = BATCH * HIST

_NC = 2
_NS = 16
_NW = _NC * _NS

_ROWS_W = BATCH // _NW
_R = 8
_CHUNK = _R * HIST
_N_CHUNKS = _ROWS_W // _R
_NBUF = 2


def _embed_kernel(idx_hbm, table_hbm, out_hbm,
                  idx_v0, idx_v1, rows_v0, rows_v1,
                  sem_i0, sem_i1, sem_g0, sem_g1, sem_o0, sem_o1):
    wid = lax.axis_index("s") * _NC + lax.axis_index("c")
    row_w = wid * _ROWS_W
    idx_v = (idx_v0, idx_v1)
    rows_v = (rows_v0, rows_v1)
    sem_i = (sem_i0, sem_i1)
    sem_g = (sem_g0, sem_g1)
    sem_o = (sem_o0, sem_o1)
    last = _N_CHUNKS - 1

    def start_idx(c, b):
        cc = jnp.minimum(c, last)
        pltpu.async_copy(idx_hbm.at[pl.ds((row_w + cc * _R) * HIST, _CHUNK)],
                         idx_v[b], sem_i[b])

    def wait_out(b):
        for k in range(_R):
            pltpu.make_async_copy(rows_v[b].at[pl.ds(k * HIST, HIST)],
                                  out_hbm.at[0], sem_o[b]).wait()

    start_idx(0, 0)
    start_idx(1, 1)

    def outer(g, carry):
        for b in range(_NBUF):
            c = g * _NBUF + b
            pltpu.make_async_copy(idx_hbm.at[pl.ds(0, _CHUNK)],
                                  idx_v[b], sem_i[b]).wait()

            @pl.when(c >= _NBUF)
            def _wait_prev_out():
                wait_out(b)

            pltpu.async_copy(table_hbm.at[idx_v[b]], rows_v[b],
                             sem_g[b]).wait()
            start_idx(c + _NBUF, b)
            row0 = row_w + c * _R
            for k in range(_R):
                pltpu.async_copy(rows_v[b].at[pl.ds(k * HIST, HIST)],
                                 out_hbm.at[row0 + k], sem_o[b])
        return carry

    lax.fori_loop(0, _N_CHUNKS // _NBUF, outer, 0)

    for b in range(_NBUF):
        pltpu.make_async_copy(idx_hbm.at[pl.ds(0, _CHUNK)],
                              idx_v[b], sem_i[b]).wait()
        wait_out(b)


@jax.jit
def _embed(a_flat, emb_weight):
    mesh = plsc.VectorSubcoreMesh(core_axis_name="c", subcore_axis_name="s")
    run = pl.kernel(
        _embed_kernel,
        out_type=jax.ShapeDtypeStruct((BATCH, HIST, A_DIM), jnp.float32),
        mesh=mesh,
        scratch_types=[
            pltpu.VMEM((_CHUNK,), jnp.int32),
            pltpu.VMEM((_CHUNK,), jnp.int32),
            pltpu.VMEM((_CHUNK, A_DIM), jnp.float32),
            pltpu.VMEM((_CHUNK, A_DIM), jnp.float32),
            pltpu.SemaphoreType.DMA,
            pltpu.SemaphoreType.DMA,
            pltpu.SemaphoreType.DMA,
            pltpu.SemaphoreType.DMA,
            pltpu.SemaphoreType.DMA,
            pltpu.SemaphoreType.DMA,
        ],
        compiler_params=pltpu.CompilerParams(use_tc_tiling_on_sc=False),
    )
    return run(a_flat, emb_weight)


def kernel(a, emb_weight):
    a_flat = a.astype(jnp.int32).reshape(B_TOTAL)
    return _embed(a_flat, emb_weight)


# speedup vs baseline: 1.0523x; 1.0523x over previous
import functools

import jax
import jax.numpy as jnp
from jax import lax
from jax.experimental import pallas as pl
from jax.experimental.pallas import tpu as pltpu
from jax.experimental.pallas import tpu_sc as plsc

N_ACTIONS = 100000
A_DIM = 16
BATCH = 16384
HIST = 200
B_TOTAL = BATCH * HIST

_NC = 2
_NS = 16
_NW = _NC * _NS

_ROWS_W = BATCH // _NW
_R = 4
_CHUNK = _R * HIST
_N_CHUNKS = _ROWS_W // _R
_NBUF = 2


_TAB_ROWS_S = N_ACTIONS // _NS


def _embed_kernel(idx_hbm, table_hbm, out_hbm,
                  table_sh, idx_v0, idx_v1, rows_v0, rows_v1,
                  sem_t, sem_i0, sem_i1, sem_g0, sem_g1, sem_o0, sem_o1):
    sid = lax.axis_index("s")
    wid = sid * _NC + lax.axis_index("c")
    row_w = wid * _ROWS_W

    slab = pl.ds(sid * _TAB_ROWS_S, _TAB_ROWS_S)
    pltpu.async_copy(table_hbm.at[slab], table_sh.at[slab], sem_t).wait()
    plsc.subcore_barrier()
    idx_v = (idx_v0, idx_v1)
    rows_v = (rows_v0, rows_v1)
    sem_i = (sem_i0, sem_i1)
    sem_g = (sem_g0, sem_g1)
    sem_o = (sem_o0, sem_o1)
    last = _N_CHUNKS - 1

    def start_idx(c, b):
        cc = jnp.minimum(c, last)
        pltpu.async_copy(idx_hbm.at[pl.ds((row_w + cc * _R) * HIST, _CHUNK)],
                         idx_v[b], sem_i[b])

    def wait_out(b):
        for k in range(_R):
            pltpu.make_async_copy(rows_v[b].at[pl.ds(k * HIST, HIST)],
                                  out_hbm.at[0], sem_o[b]).wait()

    start_idx(0, 0)
    start_idx(1, 1)

    def outer(g, carry):
        for b in range(_NBUF):
            c = g * _NBUF + b
            pltpu.make_async_copy(idx_hbm.at[pl.ds(0, _CHUNK)],
                                  idx_v[b], sem_i[b]).wait()

            @pl.when(c >= _NBUF)
            def _wait_prev_out():
                wait_out(b)

            pltpu.async_copy(table_sh.at[idx_v[b]], rows_v[b],
                             sem_g[b]).wait()
            start_idx(c + _NBUF, b)
            row0 = row_w + c * _R
            for k in range(_R):
                pltpu.async_copy(rows_v[b].at[pl.ds(k * HIST, HIST)],
                                 out_hbm.at[row0 + k], sem_o[b])
        return carry

    lax.fori_loop(0, _N_CHUNKS // _NBUF, outer, 0)

    for b in range(_NBUF):
        pltpu.make_async_copy(idx_hbm.at[pl.ds(0, _CHUNK)],
                              idx_v[b], sem_i[b]).wait()
        wait_out(b)


@jax.jit
def _embed(a_flat, emb_weight):
    mesh = plsc.VectorSubcoreMesh(core_axis_name="c", subcore_axis_name="s")
    run = pl.kernel(
        _embed_kernel,
        out_type=jax.ShapeDtypeStruct((BATCH, HIST, A_DIM), jnp.float32),
        mesh=mesh,
        scratch_types=[
            pltpu.VMEM_SHARED((N_ACTIONS, A_DIM), jnp.float32),
            pltpu.VMEM((_CHUNK,), jnp.int32),
            pltpu.VMEM((_CHUNK,), jnp.int32),
            pltpu.VMEM((_CHUNK, A_DIM), jnp.float32),
            pltpu.VMEM((_CHUNK, A_DIM), jnp.float32),
            pltpu.SemaphoreType.DMA,
            pltpu.SemaphoreType.DMA,
            pltpu.SemaphoreType.DMA,
            pltpu.SemaphoreType.DMA,
            pltpu.SemaphoreType.DMA,
            pltpu.SemaphoreType.DMA,
            pltpu.SemaphoreType.DMA,
        ],
        compiler_params=pltpu.CompilerParams(use_tc_tiling_on_sc=False),
    )
    return run(a_flat, emb_weight)


def kernel(a, emb_weight):
    a_flat = a.astype(jnp.int32).reshape(B_TOTAL)
    return _embed(a_flat, emb_weight)


# speedup vs baseline: 1.0567x; 1.0042x over previous
import functools

import jax
import jax.numpy as jnp
from jax import lax
from jax.experimental import pallas as pl
from jax.experimental.pallas import tpu as pltpu
from jax.experimental.pallas import tpu_sc as plsc

N_ACTIONS = 100000
A_DIM = 16
BATCH = 16384
HIST = 200
B_TOTAL = BATCH * HIST

_NC = 2
_NS = 16
_NW = _NC * _NS

_ROWS_W = BATCH // _NW
_R = 2
_CHUNK = _R * HIST
_N_CHUNKS = _ROWS_W // _R
_NBUF = 4
_K = 2

_TAB_ROWS_S = N_ACTIONS // _NS


def _embed_kernel(idx_hbm, table_hbm, out_hbm, table_sh,
                  idx_vs, rows_vs, sem_t, sem_is, sem_gs, sem_os):
    sid = lax.axis_index("s")
    wid = sid * _NC + lax.axis_index("c")
    row_w = wid * _ROWS_W
    last = _N_CHUNKS - 1

    slab = pl.ds(sid * _TAB_ROWS_S, _TAB_ROWS_S)
    pltpu.async_copy(table_hbm.at[slab], table_sh.at[slab], sem_t).wait()
    plsc.subcore_barrier()

    def start_idx(c, b):
        cc = jnp.minimum(c, last)
        pltpu.async_copy(idx_hbm.at[pl.ds((row_w + cc * _R) * HIST, _CHUNK)],
                         idx_vs[b], sem_is[b])

    def wait_idx(b):
        pltpu.make_async_copy(idx_hbm.at[pl.ds(0, _CHUNK)],
                              idx_vs[b], sem_is[b]).wait()

    def fire_gather(b):
        pltpu.async_copy(table_sh.at[idx_vs[b]], rows_vs[b], sem_gs[b])

    def wait_gather(b):
        pltpu.make_async_copy(table_sh.at[idx_vs[b]], rows_vs[b],
                              sem_gs[b]).wait()

    def fire_store(c, b):
        pltpu.async_copy(rows_vs[b],
                         out_hbm.at[pl.ds((row_w + c * _R) * HIST, _CHUNK)],
                         sem_os[b])

    def wait_store(b):
        pltpu.make_async_copy(rows_vs[b], out_hbm.at[pl.ds(0, _CHUNK)],
                              sem_os[b]).wait()

    for b in range(_NBUF):
        start_idx(b, b)
    for b in range(_K):
        wait_idx(b)
        fire_gather(b)

    def outer(g, carry):
        for b in range(_NBUF):
            c = g * _NBUF + b
            b2 = (b + _K) % _NBUF
            wait_gather(b)
            fire_store(c, b)
            start_idx(c + _NBUF, b)

            @pl.when(c + _K < _N_CHUNKS)
            def _fire_ahead():
                wait_idx(b2)

                @pl.when(c + _K >= _NBUF)
                def _reuse():
                    wait_store(b2)
                fire_gather(b2)
        return carry

    lax.fori_loop(0, _N_CHUNKS // _NBUF, outer, 0)

    for b in range(_NBUF):
        wait_idx(b)
        wait_store(b)


@jax.jit
def _embed(a_flat, emb_weight):
    mesh = plsc.VectorSubcoreMesh(core_axis_name="c", subcore_axis_name="s")
    run = pl.kernel(
        functools.partial(_wrapped),
        out_type=jax.ShapeDtypeStruct((B_TOTAL, A_DIM), jnp.float32),
        mesh=mesh,
        scratch_types=(
            [pltpu.VMEM_SHARED((N_ACTIONS, A_DIM), jnp.float32)]
            + [pltpu.VMEM((_CHUNK,), jnp.int32) for _ in range(_NBUF)]
            + [pltpu.VMEM((_CHUNK, A_DIM), jnp.float32) for _ in range(_NBUF)]
            + [pltpu.SemaphoreType.DMA for _ in range(1 + 3 * _NBUF)]
        ),
        compiler_params=pltpu.CompilerParams(use_tc_tiling_on_sc=False),
    )
    return run(a_flat, emb_weight)


def _wrapped(idx_hbm, table_hbm, out_hbm, table_sh, *rest):
    idx_vs = rest[0:_NBUF]
    rows_vs = rest[_NBUF:2 * _NBUF]
    sem_t = rest[2 * _NBUF]
    sem_is = rest[2 * _NBUF + 1: 3 * _NBUF + 1]
    sem_gs = rest[3 * _NBUF + 1: 4 * _NBUF + 1]
    sem_os = rest[4 * _NBUF + 1: 5 * _NBUF + 1]
    _embed_kernel(idx_hbm, table_hbm, out_hbm, table_sh,
                  idx_vs, rows_vs, sem_t, sem_is, sem_gs, sem_os)


def kernel(a, emb_weight):
    a_flat = a.astype(jnp.int32).reshape(B_TOTAL)
    return _embed(a_flat, emb_weight).reshape(BATCH, HIST, A_DIM)
